# baseline (speedup 1.0000x reference)
import functools

import jax
import jax.numpy as jnp
from jax import lax
from jax.experimental import pallas as pl
from jax.experimental.pallas import tpu as pltpu
from jax.experimental.pallas import tpu_sc as plsc

B = 16384
VOCAB = 101
EMB = 10
VPAD = 128
FPAD = 16


def _table_body(x_ref, emb_ref, ker_ref, bias_ref, scale_ref, bnb_ref, ot_ref):
    xi = x_ref[...]
    vocab = lax.broadcasted_iota(jnp.int32, (1, VPAD), 1)
    oh = (xi == vocab).astype(jnp.float32)
    counts = jnp.sum(oh, axis=0, keepdims=True)
    t = jnp.dot(emb_ref[...], ker_ref[...],
                preferred_element_type=jnp.float32) + bias_ref[...]
    mean = jnp.dot(counts, t, preferred_element_type=jnp.float32) * (1.0 / B)
    tc = t - mean
    var = jnp.dot(counts, tc * tc,
                  preferred_element_type=jnp.float32) * (1.0 / B)
    inv = lax.rsqrt(var + 1e-5)
    ot = tc * inv * scale_ref[...] + bnb_ref[...]
    nrm = jnp.sum(ot * ot, axis=1, keepdims=True)
    ot_ref[...] = ot / jnp.sqrt(nrm)


_table_call = pl.pallas_call(
    _table_body,
    out_shape=jax.ShapeDtypeStruct((VPAD, FPAD), jnp.float32),
)


def _make_gather():
    info = plsc.get_sparse_core_info()
    nc, ns = info.num_cores, info.num_subcores
    nw = nc * ns
    bw = B // nw
    mesh = plsc.VectorSubcoreMesh(core_axis_name="c", subcore_axis_name="s")

    @functools.partial(
        pl.kernel, mesh=mesh,
        compiler_params=pltpu.CompilerParams(use_tc_tiling_on_sc=False),
        out_type=jax.ShapeDtypeStruct((B, FPAD), jnp.float32),
        scratch_types=[
            pltpu.VMEM((bw,), jnp.int32),
            pltpu.VMEM((bw, FPAD), jnp.float32),
            pltpu.SemaphoreType.DMA,
        ],
    )
    def _gather(table_hbm, idx_hbm, out_hbm, idx_v, rows_v, sem):
        wid = lax.axis_index("s") * nc + lax.axis_index("c")
        base = wid * bw
        pltpu.sync_copy(idx_hbm.at[pl.ds(base, bw)], idx_v)
        pltpu.async_copy(table_hbm.at[idx_v], rows_v, sem).wait()
        pltpu.sync_copy(rows_v, out_hbm.at[pl.ds(base, bw)])

    return _gather


def kernel(x, embedding, kernel, bias, scale, bn_bias):
    xf = x.reshape(B).astype(jnp.int32)
    x2 = xf.reshape(B, 1)
    emb_p = jnp.zeros((VPAD, FPAD), jnp.float32).at[:VOCAB, :EMB].set(embedding)
    ker_p = jnp.zeros((FPAD, FPAD), jnp.float32).at[:EMB, :EMB].set(kernel)
    bias_p = jnp.zeros((1, FPAD), jnp.float32).at[0, :EMB].set(bias)
    scale_p = jnp.zeros((1, FPAD), jnp.float32).at[0, :EMB].set(scale)
    bnb_p = jnp.zeros((1, FPAD), jnp.float32).at[0, :EMB].set(bn_bias)

    out_table = _table_call(x2, emb_p, ker_p, bias_p, scale_p, bnb_p)
    rows = _make_gather()(out_table, xf)
    return rows[:, :EMB]


# speedup vs baseline: 1.0339x; 1.0339x over previous
import functools

import jax
import jax.numpy as jnp
from jax import lax
from jax.experimental import pallas as pl
from jax.experimental.pallas import tpu as pltpu
from jax.experimental.pallas import tpu_sc as plsc

B = 16384
VOCAB = 101
EMB = 10
VPAD = 128


def _table_body(x_ref, emb_ref, ker_ref, bias_ref, scale_ref, bnb_ref, ot_ref):
    xi = x_ref[...]
    vocab = lax.broadcasted_iota(jnp.int32, (1, VPAD), 1)
    oh = (xi == vocab).astype(jnp.float32)
    counts = jnp.sum(oh, axis=0, keepdims=True)[:, :VOCAB]
    t = jnp.dot(emb_ref[...], ker_ref[...],
                preferred_element_type=jnp.float32) + bias_ref[...]
    mean = jnp.dot(counts, t, preferred_element_type=jnp.float32) * (1.0 / B)
    tc = t - mean
    var = jnp.dot(counts, tc * tc,
                  preferred_element_type=jnp.float32) * (1.0 / B)
    inv = lax.rsqrt(var + 1e-5)
    ot = tc * inv * scale_ref[...] + bnb_ref[...]
    nrm = jnp.sum(ot * ot, axis=1, keepdims=True)
    ot_ref[...] = ot / jnp.sqrt(nrm)


_table_call = pl.pallas_call(
    _table_body,
    out_shape=jax.ShapeDtypeStruct((VOCAB, EMB), jnp.float32),
)


def _make_gather():
    info = plsc.get_sparse_core_info()
    nc, ns = info.num_cores, info.num_subcores
    nw = nc * ns
    bw = B // nw
    mesh = plsc.VectorSubcoreMesh(core_axis_name="c", subcore_axis_name="s")

    @functools.partial(
        pl.kernel, mesh=mesh,
        compiler_params=pltpu.CompilerParams(use_tc_tiling_on_sc=False),
        out_type=jax.ShapeDtypeStruct((B, EMB), jnp.float32),
        scratch_types=[
            pltpu.VMEM((bw,), jnp.int32),
            pltpu.VMEM((bw, EMB), jnp.float32),
            pltpu.SemaphoreType.DMA,
        ],
    )
    def _gather(table_hbm, idx_hbm, out_hbm, idx_v, rows_v, sem):
        wid = lax.axis_index("s") * nc + lax.axis_index("c")
        base = wid * bw
        pltpu.sync_copy(idx_hbm.at[pl.ds(base, bw)], idx_v)
        pltpu.async_copy(table_hbm.at[idx_v], rows_v, sem).wait()
        pltpu.sync_copy(rows_v, out_hbm.at[pl.ds(base, bw)])

    return _gather


def kernel(x, embedding, kernel, bias, scale, bn_bias):
    xf = x.reshape(B).astype(jnp.int32)
    x2 = xf.reshape(B, 1)
    out_table = _table_call(x2, embedding, kernel,
                            bias.reshape(1, EMB), scale.reshape(1, EMB),
                            bn_bias.reshape(1, EMB))
    return _make_gather()(out_table, xf)


# speedup vs baseline: 1.1247x; 1.0879x over previous
import functools

import jax
import jax.numpy as jnp
from jax import lax
from jax.experimental import pallas as pl
from jax.experimental.pallas import tpu as pltpu
from jax.experimental.pallas import tpu_sc as plsc

B = 16384
VOCAB = 101
EMB = 10
VPAD = 128
FPAD = 16


def _table_body(x_ref, emb_ref, ker_ref, bias_ref, scale_ref, bnb_ref, ot_ref):
    xi = x_ref[...]
    vocab = lax.broadcasted_iota(jnp.int32, (1, VPAD), 1)
    oh = (xi == vocab).astype(jnp.float32)
    counts = jnp.sum(oh, axis=0, keepdims=True)[:, :VOCAB]
    t = jnp.dot(emb_ref[...], ker_ref[...],
                preferred_element_type=jnp.float32) + bias_ref[...]
    mean = jnp.dot(counts, t, preferred_element_type=jnp.float32) * (1.0 / B)
    tc = t - mean
    var = jnp.dot(counts, tc * tc,
                  preferred_element_type=jnp.float32) * (1.0 / B)
    inv = lax.rsqrt(var + 1e-5)
    ot = tc * inv * scale_ref[...] + bnb_ref[...]
    nrm = jnp.sum(ot * ot, axis=1, keepdims=True)
    otn = ot / jnp.sqrt(nrm)
    ot_ref[...] = jnp.concatenate(
        [otn, jnp.zeros((VOCAB, FPAD - EMB), jnp.float32)], axis=1)


_table_call = pl.pallas_call(
    _table_body,
    out_shape=jax.ShapeDtypeStruct((VOCAB, FPAD), jnp.float32),
)


def _make_gather():
    info = plsc.get_sparse_core_info()
    nc, ns = info.num_cores, info.num_subcores
    nw = nc * ns
    bw = B // nw
    mesh = plsc.VectorSubcoreMesh(core_axis_name="c", subcore_axis_name="s")

    @functools.partial(
        pl.kernel, mesh=mesh,
        compiler_params=pltpu.CompilerParams(use_tc_tiling_on_sc=False),
        out_type=jax.ShapeDtypeStruct((B, FPAD), jnp.float32),
        scratch_types=[
            pltpu.VMEM((bw,), jnp.int32),
            pltpu.VMEM((bw, FPAD), jnp.float32),
            pltpu.SemaphoreType.DMA,
        ],
    )
    def _gather(table_hbm, idx_hbm, out_hbm, idx_v, rows_v, sem):
        wid = lax.axis_index("s") * nc + lax.axis_index("c")
        base = wid * bw
        pltpu.sync_copy(idx_hbm.at[pl.ds(base, bw)], idx_v)
        pltpu.async_copy(table_hbm.at[idx_v], rows_v, sem).wait()
        pltpu.sync_copy(rows_v, out_hbm.at[pl.ds(base, bw)])

    return _gather


def kernel(x, embedding, kernel, bias, scale, bn_bias):
    xf = x.reshape(B).astype(jnp.int32)
    x2 = xf.reshape(B, 1)
    out_table = _table_call(x2, embedding, kernel,
                            bias.reshape(1, EMB), scale.reshape(1, EMB),
                            bn_bias.reshape(1, EMB))
    return _make_gather()(out_table, xf)[:, :EMB]


# speedup vs baseline: 1.1802x; 1.0493x over previous
import functools

import jax
import jax.numpy as jnp
from jax import lax
from jax.experimental import pallas as pl
from jax.experimental.pallas import tpu as pltpu
from jax.experimental.pallas import tpu_sc as plsc

B = 16384
VOCAB = 101
EMB = 10
VPAD = 128
L = 16
NCHUNK = VPAD // L

OFF_EMB = 0
OFF_KER = OFF_EMB + VOCAB * EMB
OFF_BIAS = OFF_KER + EMB * EMB
OFF_SCALE = OFF_BIAS + EMB
OFF_BNB = OFF_SCALE + EMB
NPARAM = 1152


def _rsqrt_newton(x):
    i = lax.bitcast_convert_type(x, jnp.int32)
    y = lax.bitcast_convert_type(
        jnp.full((L,), 0x5F3759DF, jnp.int32) - (i >> 1), jnp.float32)
    for _ in range(3):
        y = y * (1.5 - 0.5 * x * y * y)
    return y


def _hsum(v, iota):
    for sh in (8, 4, 2, 1):
        v = v + jnp.take(v, iota ^ sh)
    return v


def _make_sc_kernel():
    info = plsc.get_sparse_core_info()
    nc, ns = info.num_cores, info.num_subcores
    nw = nc * ns
    bh = B // ns
    bo = B // nw
    mesh = plsc.VectorSubcoreMesh(core_axis_name="c", subcore_axis_name="s")

    @functools.partial(
        pl.kernel, mesh=mesh,
        compiler_params=pltpu.CompilerParams(
            use_tc_tiling_on_sc=False, needs_layout_passes=False),
        out_type=jax.ShapeDtypeStruct((B * EMB,), jnp.float32),
        scratch_types=[
            pltpu.VMEM((bh,), jnp.int32),
            pltpu.VMEM((NPARAM,), jnp.float32),
            pltpu.VMEM((L * VPAD,), jnp.float32),
            pltpu.VMEM((VPAD,), jnp.float32),
            pltpu.VMEM((EMB * VPAD,), jnp.float32),
            pltpu.VMEM((bo * EMB,), jnp.float32),
            pltpu.VMEM_SHARED((ns * VPAD,), jnp.float32),
        ],
    )
    def _sc_kernel(x_hbm, params_hbm, out_hbm,
                   idx_v, par_v, hist_v, cnt_v, tab_v, out_v, shared):
        c = lax.axis_index("c")
        s = lax.axis_index("s")
        wid = s * nc + c
        iota = lax.iota(jnp.int32, L)

        pltpu.sync_copy(x_hbm.at[pl.ds(s * bh, bh)], idx_v)
        pltpu.sync_copy(params_hbm, par_v)

        zeros = jnp.zeros((L,), jnp.float32)
        ones = jnp.ones((L,), jnp.float32)

        def _zero(i, _):
            hist_v[pl.ds(i * L, L)] = zeros
            return 0
        lax.fori_loop(0, L * VPAD // L, _zero, 0)

        lane_base = iota * VPAD

        def _hist(i, _):
            xv = idx_v[pl.ds(i * L, L)]
            plsc.addupdate_scatter(hist_v, [lane_base + xv], ones)
            return 0
        lax.fori_loop(0, bh // L, _hist, 0)

        def _reduce_rows(dst):
            for ch in range(NCHUNK):
                acc = hist_v[pl.ds(ch * L, L)]
                for r in range(1, ns):
                    acc = acc + hist_v[pl.ds(r * VPAD + ch * L, L)]
                dst[pl.ds(ch * L, L)] = acc
        _reduce_rows(cnt_v)

        pltpu.sync_copy(cnt_v, shared.at[pl.ds(s * VPAD, VPAD)])
        plsc.subcore_barrier()
        pltpu.sync_copy(shared, hist_v)
        _reduce_rows(cnt_v)

        def _splat(off):
            return plsc.load_gather(par_v, [jnp.full((L,), off, jnp.int32)])

        ker_s = [[_splat(OFF_KER + k * EMB + j) for j in range(EMB)]
                 for k in range(EMB)]
        bias_s = [_splat(OFF_BIAS + j) for j in range(EMB)]
        scale_s = [_splat(OFF_SCALE + j) for j in range(EMB)]
        bnb_s = [_splat(OFF_BNB + j) for j in range(EMB)]

        for ch in range(NCHUNK):
            vv = iota + ch * L
            vmask = vv < VOCAB
            eidx = vv * EMB
            evec = []
            for k in range(EMB):
                idx_c = jnp.minimum(eidx + k, NPARAM - 1)
                ev = plsc.load_gather(par_v, [idx_c])
                evec.append(jnp.where(vmask, ev, 0.0))
            for j in range(EMB):
                acc = bias_s[j]
                for k in range(EMB):
                    acc = acc + evec[k] * ker_s[k][j]
                tab_v[pl.ds(j * VPAD + ch * L, L)] = acc

        cvec = [cnt_v[pl.ds(ch * L, L)] for ch in range(NCHUNK)]
        inv_b = 1.0 / B
        for j in range(EMB):
            tvec = [tab_v[pl.ds(j * VPAD + ch * L, L)] for ch in range(NCHUNK)]
            macc = cvec[0] * tvec[0]
            for ch in range(1, NCHUNK):
                macc = macc + cvec[ch] * tvec[ch]
            mj = _hsum(macc, iota) * inv_b
            vacc = None
            for ch in range(NCHUNK):
                d = tvec[ch] - mj
                term = cvec[ch] * d * d
                vacc = term if vacc is None else vacc + term
            vj = _hsum(vacc, iota) * inv_b
            aj = _rsqrt_newton(vj + 1e-5) * scale_s[j]
            cj = bnb_s[j] - mj * aj
            for ch in range(NCHUNK):
                tab_v[pl.ds(j * VPAD + ch * L, L)] = tvec[ch] * aj + cj

        for ch in range(NCHUNK):
            fvec = [tab_v[pl.ds(j * VPAD + ch * L, L)] for j in range(EMB)]
            nsq = fvec[0] * fvec[0]
            for j in range(1, EMB):
                nsq = nsq + fvec[j] * fvec[j]
            rn = _rsqrt_newton(nsq)
            for j in range(EMB):
                tab_v[pl.ds(j * VPAD + ch * L, L)] = fvec[j] * rn

        out_off = c * bo

        def _emit(r, _):
            xv = idx_v[pl.ds(out_off + r * L, L)]
            obase = r * (L * EMB) + iota * EMB
            for j in range(EMB):
                vals = plsc.load_gather(tab_v, [xv + j * VPAD])
                plsc.store_scatter(out_v, [obase + j], vals)
            return 0
        lax.fori_loop(0, bo // L, _emit, 0)

        pltpu.sync_copy(out_v, out_hbm.at[pl.ds(wid * bo * EMB, bo * EMB)])

    return _sc_kernel


def kernel(x, embedding, kernel, bias, scale, bn_bias):
    xf = x.reshape(B).astype(jnp.int32)
    params = jnp.concatenate([
        embedding.reshape(VOCAB * EMB), kernel.reshape(EMB * EMB),
        bias, scale, bn_bias,
        jnp.zeros((NPARAM - OFF_BNB - EMB,), jnp.float32),
    ])
    out = _make_sc_kernel()(xf, params)
    return out.reshape(B, EMB)


# speedup vs baseline: 1.1831x; 1.0024x over previous
import functools

import jax
import jax.numpy as jnp
from jax import lax
from jax.experimental import pallas as pl
from jax.experimental.pallas import tpu as pltpu
from jax.experimental.pallas import tpu_sc as plsc

B = 16384
VOCAB = 101
EMB = 10
VPAD = 128
L = 16
NCHUNK = VPAD // L
NEMB = VOCAB * EMB


def _rsqrt_newton(x):
    i = lax.bitcast_convert_type(x, jnp.int32)
    y = lax.bitcast_convert_type(
        jnp.full((L,), 0x5F3759DF, jnp.int32) - (i >> 1), jnp.float32)
    for _ in range(3):
        y = y * (1.5 - 0.5 * x * y * y)
    return y


def _hsum(v, iota):
    for sh in (8, 4, 2, 1):
        v = v + jnp.take(v, iota ^ sh)
    return v


def _make_sc_kernel():
    info = plsc.get_sparse_core_info()
    nc, ns = info.num_cores, info.num_subcores
    nw = nc * ns
    bh = B // ns
    bo = B // nw
    mesh = plsc.VectorSubcoreMesh(core_axis_name="c", subcore_axis_name="s")

    @functools.partial(
        pl.kernel, mesh=mesh,
        compiler_params=pltpu.CompilerParams(
            use_tc_tiling_on_sc=False, needs_layout_passes=False),
        out_type=jax.ShapeDtypeStruct((B * EMB,), jnp.float32),
        scratch_types=[
            pltpu.VMEM((bh,), jnp.int32),
            pltpu.VMEM((NEMB,), jnp.float32),
            pltpu.VMEM((EMB * EMB,), jnp.float32),
            pltpu.VMEM((EMB,), jnp.float32),
            pltpu.VMEM((EMB,), jnp.float32),
            pltpu.VMEM((EMB,), jnp.float32),
            pltpu.VMEM((L * VPAD,), jnp.float32),
            pltpu.VMEM((VPAD,), jnp.float32),
            pltpu.VMEM((EMB * VPAD,), jnp.float32),
            pltpu.VMEM((bo * EMB,), jnp.float32),
            pltpu.VMEM_SHARED((ns * VPAD,), jnp.float32),
            pltpu.SemaphoreType.DMA,
            pltpu.SemaphoreType.DMA,
        ],
    )
    def _sc_kernel(x_hbm, emb_hbm, ker_hbm, bias_hbm, scale_hbm, bnb_hbm,
                   out_hbm, idx_v, emb_v, ker_v, bias_v, scale_v, bnb_v,
                   hist_v, cnt_v, tab_v, out_v, shared, sem_i, sem_p):
        c = lax.axis_index("c")
        s = lax.axis_index("s")
        wid = s * nc + c
        iota = lax.iota(jnp.int32, L)

        cp_idx = pltpu.async_copy(x_hbm.at[pl.ds(s * bh, bh)], idx_v, sem_i)
        cps = [pltpu.async_copy(src, dst, sem_p) for src, dst in (
            (emb_hbm, emb_v), (ker_hbm, ker_v), (bias_hbm, bias_v),
            (scale_hbm, scale_v), (bnb_hbm, bnb_v))]

        zeros = jnp.zeros((L,), jnp.float32)
        ones = jnp.ones((L,), jnp.float32)
        for i in range(L * VPAD // L):
            hist_v[pl.ds(i * L, L)] = zeros

        cp_idx.wait()

        lane_base = iota * VPAD

        def _hist(i, _):
            base = i * (4 * L)
            for u in range(4):
                xv = idx_v[pl.ds(base + u * L, L)]
                plsc.addupdate_scatter(hist_v, [lane_base + xv], ones)
            return 0
        lax.fori_loop(0, bh // (4 * L), _hist, 0)

        def _reduce_rows(dst):
            for ch in range(NCHUNK):
                acc = hist_v[pl.ds(ch * L, L)]
                for r in range(1, ns):
                    acc = acc + hist_v[pl.ds(r * VPAD + ch * L, L)]
                dst[pl.ds(ch * L, L)] = acc
        _reduce_rows(cnt_v)

        pltpu.sync_copy(cnt_v, shared.at[pl.ds(s * VPAD, VPAD)])
        plsc.subcore_barrier()
        pltpu.sync_copy(shared, hist_v)
        _reduce_rows(cnt_v)

        for cp in cps:
            cp.wait()

        def _splat(ref, off):
            return plsc.load_gather(ref, [jnp.full((L,), off, jnp.int32)])

        ker_s = [[_splat(ker_v, k * EMB + j) for j in range(EMB)]
                 for k in range(EMB)]
        bias_s = [_splat(bias_v, j) for j in range(EMB)]
        scale_s = [_splat(scale_v, j) for j in range(EMB)]
        bnb_s = [_splat(bnb_v, j) for j in range(EMB)]

        for ch in range(NCHUNK):
            vv = iota + ch * L
            vmask = vv < VOCAB
            eidx = vv * EMB
            evec = []
            for k in range(EMB):
                idx_c = jnp.minimum(eidx + k, NEMB - 1)
                ev = plsc.load_gather(emb_v, [idx_c])
                evec.append(jnp.where(vmask, ev, 0.0))
            for j in range(EMB):
                acc = bias_s[j]
                for k in range(EMB):
                    acc = acc + evec[k] * ker_s[k][j]
                tab_v[pl.ds(j * VPAD + ch * L, L)] = acc

        cvec = [cnt_v[pl.ds(ch * L, L)] for ch in range(NCHUNK)]
        inv_b = 1.0 / B
        for j in range(EMB):
            tvec = [tab_v[pl.ds(j * VPAD + ch * L, L)] for ch in range(NCHUNK)]
            macc = cvec[0] * tvec[0]
            for ch in range(1, NCHUNK):
                macc = macc + cvec[ch] * tvec[ch]
            mj = _hsum(macc, iota) * inv_b
            vacc = None
            for ch in range(NCHUNK):
                d = tvec[ch] - mj
                term = cvec[ch] * d * d
                vacc = term if vacc is None else vacc + term
            vj = _hsum(vacc, iota) * inv_b
            aj = _rsqrt_newton(vj + 1e-5) * scale_s[j]
            cj = bnb_s[j] - mj * aj
            for ch in range(NCHUNK):
                tab_v[pl.ds(j * VPAD + ch * L, L)] = tvec[ch] * aj + cj

        for ch in range(NCHUNK):
            fvec = [tab_v[pl.ds(j * VPAD + ch * L, L)] for j in range(EMB)]
            nsq = fvec[0] * fvec[0]
            for j in range(1, EMB):
                nsq = nsq + fvec[j] * fvec[j]
            rn = _rsqrt_newton(nsq)
            for j in range(EMB):
                tab_v[pl.ds(j * VPAD + ch * L, L)] = fvec[j] * rn

        out_off = c * bo

        def _emit(r, _):
            xv = idx_v[pl.ds(out_off + r * L, L)]
            obase = r * (L * EMB) + iota * EMB
            for j in range(EMB):
                vals = plsc.load_gather(tab_v, [xv + j * VPAD])
                plsc.store_scatter(out_v, [obase + j], vals)
            return 0
        lax.fori_loop(0, bo // L, _emit, 0)

        pltpu.sync_copy(out_v, out_hbm.at[pl.ds(wid * bo * EMB, bo * EMB)])

    return _sc_kernel


def kernel(x, embedding, kernel, bias, scale, bn_bias):
    xf = x.reshape(B).astype(jnp.int32)
    out = _make_sc_kernel()(
        xf, embedding.reshape(NEMB), kernel.reshape(EMB * EMB),
        bias, scale, bn_bias)
    return out.reshape(B, EMB)


# speedup vs baseline: 1.1935x; 1.0088x over previous
import functools

import jax
import jax.numpy as jnp
from jax import lax
from jax.experimental import pallas as pl
from jax.experimental.pallas import tpu as pltpu
from jax.experimental.pallas import tpu_sc as plsc

B = 16384
VOCAB = 101
EMB = 10
VPAD = 128
L = 16
NCHUNK = VPAD // L

OFF_EMB = 0
OFF_KER = OFF_EMB + VOCAB * EMB
OFF_BIAS = OFF_KER + EMB * EMB
OFF_SCALE = OFF_BIAS + EMB
OFF_BNB = OFF_SCALE + EMB
NPARAM = 1152


def _rsqrt_newton(x):
    i = lax.bitcast_convert_type(x, jnp.int32)
    y = lax.bitcast_convert_type(
        jnp.full((L,), 0x5F3759DF, jnp.int32) - (i >> 1), jnp.float32)
    for _ in range(3):
        y = y * (1.5 - 0.5 * x * y * y)
    return y


def _hsum(v, iota):
    for sh in (8, 4, 2, 1):
        v = v + jnp.take(v, iota ^ sh)
    return v


def _make_sc_kernel():
    info = plsc.get_sparse_core_info()
    nc, ns = info.num_cores, info.num_subcores
    nw = nc * ns
    bh = B // ns
    bo = B // nw
    mesh = plsc.VectorSubcoreMesh(core_axis_name="c", subcore_axis_name="s")

    @functools.partial(
        pl.kernel, mesh=mesh,
        compiler_params=pltpu.CompilerParams(
            use_tc_tiling_on_sc=False, needs_layout_passes=False),
        out_type=jax.ShapeDtypeStruct((B * EMB,), jnp.float32),
        scratch_types=[
            pltpu.VMEM((bh,), jnp.int32),
            pltpu.VMEM((NPARAM,), jnp.float32),
            pltpu.VMEM((L * VPAD,), jnp.float32),
            pltpu.VMEM((VPAD,), jnp.float32),
            pltpu.VMEM((EMB * VPAD,), jnp.float32),
            pltpu.VMEM((bo * EMB,), jnp.float32),
            pltpu.VMEM_SHARED((ns * VPAD,), jnp.float32),
            pltpu.SemaphoreType.DMA,
            pltpu.SemaphoreType.DMA,
        ],
    )
    def _sc_kernel(x_hbm, params_hbm, out_hbm, idx_v, par_v,
                   hist_v, cnt_v, tab_v, out_v, shared, sem_i, sem_p):
        c = lax.axis_index("c")
        s = lax.axis_index("s")
        wid = s * nc + c
        iota = lax.iota(jnp.int32, L)

        cp_idx = pltpu.async_copy(x_hbm.at[pl.ds(s * bh, bh)], idx_v, sem_i)
        cp_par = pltpu.async_copy(params_hbm, par_v, sem_p)

        zeros = jnp.zeros((L,), jnp.float32)
        ones = jnp.ones((L,), jnp.float32)
        for i in range(L * VPAD // L):
            hist_v[pl.ds(i * L, L)] = zeros

        cp_idx.wait()

        lane_base = iota * VPAD

        def _hist(i, _):
            base = i * (4 * L)
            for u in range(4):
                xv = idx_v[pl.ds(base + u * L, L)]
                plsc.addupdate_scatter(hist_v, [lane_base + xv], ones)
            return 0
        lax.fori_loop(0, bh // (4 * L), _hist, 0)

        def _reduce_rows(dst):
            for ch in range(NCHUNK):
                acc = hist_v[pl.ds(ch * L, L)]
                for r in range(1, ns):
                    acc = acc + hist_v[pl.ds(r * VPAD + ch * L, L)]
                dst[pl.ds(ch * L, L)] = acc
        _reduce_rows(cnt_v)

        pltpu.sync_copy(cnt_v, shared.at[pl.ds(s * VPAD, VPAD)])
        plsc.subcore_barrier()
        pltpu.sync_copy(shared, hist_v)
        _reduce_rows(cnt_v)

        cp_par.wait()

        def _splat(off):
            return plsc.load_gather(par_v, [jnp.full((L,), off, jnp.int32)])

        ker_s = [[_splat(OFF_KER + k * EMB + j) for j in range(EMB)]
                 for k in range(EMB)]
        bias_s = [_splat(OFF_BIAS + j) for j in range(EMB)]
        scale_s = [_splat(OFF_SCALE + j) for j in range(EMB)]
        bnb_s = [_splat(OFF_BNB + j) for j in range(EMB)]

        for ch in range(NCHUNK):
            vv = iota + ch * L
            vmask = vv < VOCAB
            eidx = vv * EMB
            evec = []
            for k in range(EMB):
                idx_c = jnp.minimum(eidx + k, NPARAM - 1)
                ev = plsc.load_gather(par_v, [idx_c])
                evec.append(jnp.where(vmask, ev, 0.0))
            for j in range(EMB):
                acc = bias_s[j]
                for k in range(EMB):
                    acc = acc + evec[k] * ker_s[k][j]
                tab_v[pl.ds(j * VPAD + ch * L, L)] = acc

        cvec = [cnt_v[pl.ds(ch * L, L)] for ch in range(NCHUNK)]
        inv_b = 1.0 / B
        for j in range(EMB):
            tvec = [tab_v[pl.ds(j * VPAD + ch * L, L)] for ch in range(NCHUNK)]
            macc = cvec[0] * tvec[0]
            for ch in range(1, NCHUNK):
                macc = macc + cvec[ch] * tvec[ch]
            mj = _hsum(macc, iota) * inv_b
            vacc = None
            for ch in range(NCHUNK):
                d = tvec[ch] - mj
                term = cvec[ch] * d * d
                vacc = term if vacc is None else vacc + term
            vj = _hsum(vacc, iota) * inv_b
            aj = _rsqrt_newton(vj + 1e-5) * scale_s[j]
            cj = bnb_s[j] - mj * aj
            for ch in range(NCHUNK):
                tab_v[pl.ds(j * VPAD + ch * L, L)] = tvec[ch] * aj + cj

        for ch in range(NCHUNK):
            fvec = [tab_v[pl.ds(j * VPAD + ch * L, L)] for j in range(EMB)]
            nsq = fvec[0] * fvec[0]
            for j in range(1, EMB):
                nsq = nsq + fvec[j] * fvec[j]
            rn = _rsqrt_newton(nsq)
            for j in range(EMB):
                tab_v[pl.ds(j * VPAD + ch * L, L)] = fvec[j] * rn

        out_off = c * bo

        def _emit(r, _):
            xv = idx_v[pl.ds(out_off + r * L, L)]
            obase = r * (L * EMB) + iota * EMB
            for j in range(EMB):
                vals = plsc.load_gather(tab_v, [xv + j * VPAD])
                plsc.store_scatter(out_v, [obase + j], vals)
            return 0
        lax.fori_loop(0, bo // L, _emit, 0)

        pltpu.sync_copy(out_v, out_hbm.at[pl.ds(wid * bo * EMB, bo * EMB)])

    return _sc_kernel


def kernel(x, embedding, kernel, bias, scale, bn_bias):
    xf = x.reshape(B).astype(jnp.int32)
    params = jnp.concatenate([
        embedding.reshape(VOCAB * EMB), kernel.reshape(EMB * EMB),
        bias, scale, bn_bias,
        jnp.zeros((NPARAM - OFF_BNB - EMB,), jnp.float32),
    ])
    out = _make_sc_kernel()(xf, params)
    return out.reshape(B, EMB)


# speedup vs baseline: 1.2268x; 1.0279x over previous
import functools

import jax
import jax.numpy as jnp
from jax import lax
from jax.experimental import pallas as pl
from jax.experimental.pallas import tpu as pltpu
from jax.experimental.pallas import tpu_sc as plsc

B = 16384
VOCAB = 101
EMB = 10
VPAD = 128
L = 16
NCHUNK = VPAD // L

OFF_EMB = 0
OFF_KER = OFF_EMB + VOCAB * EMB
OFF_BIAS = OFF_KER + EMB * EMB
OFF_SCALE = OFF_BIAS + EMB
OFF_BNB = OFF_SCALE + EMB
NPARAM = 1152


def _rsqrt_newton(x):
    i = lax.bitcast_convert_type(x, jnp.int32)
    y = lax.bitcast_convert_type(
        jnp.full((L,), 0x5F3759DF, jnp.int32) - (i >> 1), jnp.float32)
    for _ in range(3):
        y = y * (1.5 - 0.5 * x * y * y)
    return y


def _hsum(v, iota):
    for sh in (8, 4, 2, 1):
        v = v + jnp.take(v, iota ^ sh)
    return v


def _make_sc_kernel():
    info = plsc.get_sparse_core_info()
    nc, ns = info.num_cores, info.num_subcores
    nw = nc * ns
    bh = B // ns
    bo = B // nw
    mesh = plsc.VectorSubcoreMesh(core_axis_name="c", subcore_axis_name="s")

    @functools.partial(
        pl.kernel, mesh=mesh,
        compiler_params=pltpu.CompilerParams(
            use_tc_tiling_on_sc=False, needs_layout_passes=False),
        out_type=jax.ShapeDtypeStruct((B * EMB,), jnp.float32),
        scratch_types=[
            pltpu.VMEM((bh,), jnp.int32),
            pltpu.VMEM((NPARAM,), jnp.float32),
            pltpu.VMEM((L * VPAD,), jnp.float32),
            pltpu.VMEM((VPAD,), jnp.float32),
            pltpu.VMEM((EMB * VPAD,), jnp.float32),
            pltpu.VMEM((bo * EMB,), jnp.float32),
            pltpu.VMEM_SHARED((ns * VPAD,), jnp.float32),
            pltpu.SemaphoreType.DMA,
            pltpu.SemaphoreType.DMA,
        ],
    )
    def _sc_kernel(x_hbm, params_hbm, out_hbm, idx_v, par_v,
                   hist_v, cnt_v, tab_v, out_v, shared, sem_i, sem_p):
        c = lax.axis_index("c")
        s = lax.axis_index("s")
        wid = s * nc + c
        iota = lax.iota(jnp.int32, L)

        cp_idx = pltpu.async_copy(x_hbm.at[pl.ds(s * bh, bh)], idx_v, sem_i)
        cp_par = pltpu.async_copy(params_hbm, par_v, sem_p)

        zeros = jnp.zeros((L,), jnp.float32)
        ones = jnp.ones((L,), jnp.float32)

        def _zero(i, _):
            hist_v[pl.ds(i * L, L)] = zeros
            return 0
        lax.fori_loop(0, L * VPAD // L, _zero, 0)

        cp_idx.wait()

        lane_base = iota * VPAD

        def _hist(i, _):
            xv = idx_v[pl.ds(i * L, L)]
            plsc.addupdate_scatter(hist_v, [lane_base + xv], ones)
            return 0
        lax.fori_loop(0, bh // L, _hist, 0)

        def _reduce_rows(dst):
            for ch in range(NCHUNK):
                acc = hist_v[pl.ds(ch * L, L)]
                for r in range(1, ns):
                    acc = acc + hist_v[pl.ds(r * VPAD + ch * L, L)]
                dst[pl.ds(ch * L, L)] = acc
        _reduce_rows(cnt_v)

        pltpu.sync_copy(cnt_v, shared.at[pl.ds(s * VPAD, VPAD)])
        plsc.subcore_barrier()
        pltpu.sync_copy(shared, hist_v)
        _reduce_rows(cnt_v)

        cp_par.wait()

        def _splat(off):
            return plsc.load_gather(par_v, [jnp.full((L,), off, jnp.int32)])

        ker_s = [[_splat(OFF_KER + k * EMB + j) for j in range(EMB)]
                 for k in range(EMB)]
        bias_s = [_splat(OFF_BIAS + j) for j in range(EMB)]

        for ch in range(NCHUNK - 1):
            vv = iota + ch * L
            vmask = vv < VOCAB
            eidx = vv * EMB
            evec = []
            for k in range(EMB):
                idx_c = jnp.minimum(eidx + k, NPARAM - 1)
                ev = plsc.load_gather(par_v, [idx_c])
                evec.append(jnp.where(vmask, ev, 0.0))
            for j in range(EMB):
                acc = bias_s[j]
                for k in range(EMB):
                    acc = acc + evec[k] * ker_s[k][j]
                tab_v[pl.ds(j * VPAD + ch * L, L)] = acc
        for j in range(EMB):
            tab_v[pl.ds(j * VPAD + (NCHUNK - 1) * L, L)] = zeros

        cvec = [cnt_v[pl.ds(ch * L, L)] for ch in range(NCHUNK)]
        inv_b = 1.0 / B

        def _stats(j, _):
            jb = j * VPAD
            tvec = [tab_v[pl.ds(jb + ch * L, L)] for ch in range(NCHUNK)]
            macc = cvec[0] * tvec[0]
            for ch in range(1, NCHUNK):
                macc = macc + cvec[ch] * tvec[ch]
            mj = _hsum(macc, iota) * inv_b
            vacc = None
            for ch in range(NCHUNK):
                d = tvec[ch] - mj
                term = cvec[ch] * d * d
                vacc = term if vacc is None else vacc + term
            vj = _hsum(vacc, iota) * inv_b
            sc_j = plsc.load_gather(
                par_v, [jnp.full((L,), OFF_SCALE, jnp.int32) + j])
            bn_j = plsc.load_gather(
                par_v, [jnp.full((L,), OFF_BNB, jnp.int32) + j])
            aj = _rsqrt_newton(vj + 1e-5) * sc_j
            cj = bn_j - mj * aj
            for ch in range(NCHUNK):
                tab_v[pl.ds(jb + ch * L, L)] = tvec[ch] * aj + cj
            return 0
        lax.fori_loop(0, EMB, _stats, 0)

        def _norm(ch, _):
            cb = ch * L
            fvec = [tab_v[pl.ds(j * VPAD + cb, L)] for j in range(EMB)]
            nsq = fvec[0] * fvec[0]
            for j in range(1, EMB):
                nsq = nsq + fvec[j] * fvec[j]
            rn = _rsqrt_newton(nsq)
            for j in range(EMB):
                tab_v[pl.ds(j * VPAD + cb, L)] = fvec[j] * rn
            return 0
        lax.fori_loop(0, NCHUNK - 1, _norm, 0)

        out_off = c * bo

        def _emit(r, _):
            xv = idx_v[pl.ds(out_off + r * L, L)]
            obase = r * (L * EMB) + iota * EMB
            for j in range(EMB):
                vals = plsc.load_gather(tab_v, [xv + j * VPAD])
                plsc.store_scatter(out_v, [obase + j], vals)
            return 0
        lax.fori_loop(0, bo // L, _emit, 0)

        pltpu.sync_copy(out_v, out_hbm.at[pl.ds(wid * bo * EMB, bo * EMB)])

    return _sc_kernel


def kernel(x, embedding, kernel, bias, scale, bn_bias):
    xf = x.reshape(B).astype(jnp.int32)
    params = jnp.concatenate([
        embedding.reshape(VOCAB * EMB), kernel.reshape(EMB * EMB),
        bias, scale, bn_bias,
        jnp.zeros((NPARAM - OFF_BNB - EMB,), jnp.float32),
    ])
    out = _make_sc_kernel()(xf, params)
    return out.reshape(B, EMB)


# speedup vs baseline: 1.2382x; 1.0093x over previous
import functools

import jax
import jax.numpy as jnp
from jax import lax
from jax.experimental import pallas as pl
from jax.experimental.pallas import tpu as pltpu
from jax.experimental.pallas import tpu_sc as plsc

B = 16384
VOCAB = 101
EMB = 10
VPAD = 128
L = 16
NCHUNK = VPAD // L

OFF_EMB = 0
OFF_KER = OFF_EMB + VOCAB * EMB
OFF_BIAS = OFF_KER + EMB * EMB
OFF_SCALE = OFF_BIAS + EMB
OFF_BNB = OFF_SCALE + EMB
NPARAM = 1152


def _rsqrt_newton(x):
    i = lax.bitcast_convert_type(x, jnp.int32)
    y = lax.bitcast_convert_type(
        jnp.full((L,), 0x5F3759DF, jnp.int32) - (i >> 1), jnp.float32)
    for _ in range(3):
        y = y * (1.5 - 0.5 * x * y * y)
    return y


def _hsum(v, iota):
    for sh in (8, 4, 2, 1):
        v = v + jnp.take(v, iota ^ sh)
    return v


def _make_sc_kernel():
    info = plsc.get_sparse_core_info()
    nc, ns = info.num_cores, info.num_subcores
    nw = nc * ns
    bh = B // ns
    bo = B // nw
    mesh = plsc.VectorSubcoreMesh(core_axis_name="c", subcore_axis_name="s")

    @functools.partial(
        pl.kernel, mesh=mesh,
        compiler_params=pltpu.CompilerParams(
            use_tc_tiling_on_sc=False, needs_layout_passes=False),
        out_type=jax.ShapeDtypeStruct((B * EMB,), jnp.float32),
        scratch_types=[
            pltpu.VMEM((bh,), jnp.int32),
            pltpu.VMEM((NPARAM,), jnp.float32),
            pltpu.VMEM((L * VPAD,), jnp.float32),
            pltpu.VMEM((VPAD,), jnp.float32),
            pltpu.VMEM((EMB * VPAD,), jnp.float32),
            pltpu.VMEM((bo * EMB,), jnp.float32),
            pltpu.VMEM_SHARED((ns * VPAD,), jnp.float32),
            pltpu.SemaphoreType.DMA,
            pltpu.SemaphoreType.DMA,
        ],
    )
    def _sc_kernel(x_hbm, params_hbm, out_hbm, idx_v, par_v,
                   hist_v, cnt_v, tab_v, out_v, shared, sem_i, sem_p):
        c = lax.axis_index("c")
        s = lax.axis_index("s")
        wid = s * nc + c
        iota = lax.iota(jnp.int32, L)

        cp_idx = pltpu.async_copy(x_hbm.at[pl.ds(s * bh, bh)], idx_v, sem_i)
        cp_par = pltpu.async_copy(params_hbm, par_v, sem_p)

        zeros = jnp.zeros((L,), jnp.float32)
        ones = jnp.ones((L,), jnp.float32)

        def _zero(i, _):
            hist_v[pl.ds(i * L, L)] = zeros
            return 0
        lax.fori_loop(0, L * VPAD // L, _zero, 0)

        cp_idx.wait()

        lane_base = iota * VPAD

        def _hist(i, _):
            xv = idx_v[pl.ds(i * L, L)]
            plsc.addupdate_scatter(hist_v, [lane_base + xv], ones)
            return 0
        lax.fori_loop(0, bh // L, _hist, 0)

        def _reduce_rows():
            def body(ch, _):
                cb = ch * L
                acc = hist_v[pl.ds(cb, L)]
                for r in range(1, ns):
                    acc = acc + hist_v[pl.ds(r * VPAD + cb, L)]
                cnt_v[pl.ds(cb, L)] = acc
                return 0
            lax.fori_loop(0, NCHUNK, body, 0)
        _reduce_rows()

        pltpu.sync_copy(cnt_v, shared.at[pl.ds(s * VPAD, VPAD)])
        plsc.subcore_barrier()
        pltpu.sync_copy(shared, hist_v)
        _reduce_rows()

        cp_par.wait()

        def _splat(off):
            return plsc.load_gather(par_v, [jnp.full((L,), off, jnp.int32)])

        ker_s = [[_splat(OFF_KER + k * EMB + j) for j in range(EMB)]
                 for k in range(EMB)]
        bias_s = [_splat(OFF_BIAS + j) for j in range(EMB)]

        for ch in range(NCHUNK - 1):
            vv = iota + ch * L
            vmask = vv < VOCAB
            eidx = vv * EMB
            evec = []
            for k in range(EMB):
                idx_c = jnp.minimum(eidx + k, NPARAM - 1)
                ev = plsc.load_gather(par_v, [idx_c])
                evec.append(jnp.where(vmask, ev, 0.0))
            for j in range(EMB):
                acc = bias_s[j]
                for k in range(EMB):
                    acc = acc + evec[k] * ker_s[k][j]
                tab_v[pl.ds(j * VPAD + ch * L, L)] = acc
        for j in range(EMB):
            tab_v[pl.ds(j * VPAD + (NCHUNK - 1) * L, L)] = zeros

        cvec = [cnt_v[pl.ds(ch * L, L)] for ch in range(NCHUNK)]
        inv_b = 1.0 / B

        def _stats(j, _):
            jb = j * VPAD
            tvec = [tab_v[pl.ds(jb + ch * L, L)] for ch in range(NCHUNK)]
            macc = cvec[0] * tvec[0]
            for ch in range(1, NCHUNK):
                macc = macc + cvec[ch] * tvec[ch]
            mj = _hsum(macc, iota) * inv_b
            vacc = None
            for ch in range(NCHUNK):
                d = tvec[ch] - mj
                term = cvec[ch] * d * d
                vacc = term if vacc is None else vacc + term
            vj = _hsum(vacc, iota) * inv_b
            sc_j = plsc.load_gather(
                par_v, [jnp.full((L,), OFF_SCALE, jnp.int32) + j])
            bn_j = plsc.load_gather(
                par_v, [jnp.full((L,), OFF_BNB, jnp.int32) + j])
            aj = _rsqrt_newton(vj + 1e-5) * sc_j
            cj = bn_j - mj * aj
            for ch in range(NCHUNK):
                tab_v[pl.ds(jb + ch * L, L)] = tvec[ch] * aj + cj
            return 0
        lax.fori_loop(0, EMB, _stats, 0)

        def _norm(ch, _):
            cb = ch * L
            fvec = [tab_v[pl.ds(j * VPAD + cb, L)] for j in range(EMB)]
            nsq = fvec[0] * fvec[0]
            for j in range(1, EMB):
                nsq = nsq + fvec[j] * fvec[j]
            rn = _rsqrt_newton(nsq)
            for j in range(EMB):
                tab_v[pl.ds(j * VPAD + cb, L)] = fvec[j] * rn
            return 0
        lax.fori_loop(0, NCHUNK - 1, _norm, 0)

        out_off = c * bo

        def _emit(r, _):
            xv = idx_v[pl.ds(out_off + r * L, L)]
            obase = r * (L * EMB) + iota * EMB
            for j in range(EMB):
                vals = plsc.load_gather(tab_v, [xv + j * VPAD])
                plsc.store_scatter(out_v, [obase + j], vals)
            return 0

        half = bo * EMB // 2
        lax.fori_loop(0, bo // (2 * L), _emit, 0)
        cp_lo = pltpu.async_copy(
            out_v.at[pl.ds(0, half)],
            out_hbm.at[pl.ds(wid * bo * EMB, half)], sem_i)
        lax.fori_loop(bo // (2 * L), bo // L, _emit, 0)
        cp_hi = pltpu.async_copy(
            out_v.at[pl.ds(half, half)],
            out_hbm.at[pl.ds(wid * bo * EMB + half, half)], sem_p)
        cp_lo.wait()
        cp_hi.wait()

    return _sc_kernel


def kernel(x, embedding, kernel, bias, scale, bn_bias):
    xf = x.reshape(B).astype(jnp.int32)
    params = jnp.concatenate([
        embedding.reshape(VOCAB * EMB), kernel.reshape(EMB * EMB),
        bias, scale, bn_bias,
        jnp.zeros((NPARAM - OFF_BNB - EMB,), jnp.float32),
    ])
    out = _make_sc_kernel()(xf, params)
    return out.reshape(B, EMB)
